# baseline (speedup 1.0000x reference)
import functools

import jax
import jax.numpy as jnp
from jax import lax
from jax.experimental import pallas as pl
from jax.experimental.pallas import tpu as pltpu
from jax.experimental.pallas import tpu_sc as plsc

E = 64
K = 2
H = 768
FF = 128
T = 2048
P = T * K
R = 128
NB = P // R + E
NPAD = NB * R

NC, NS, L = 2, 16, 16
NW = NC * NS

F32 = jnp.float32
BF16 = jnp.bfloat16
I32 = jnp.int32

@functools.cache
def _sc_mesh():
    return plsc.VectorSubcoreMesh(core_axis_name="c", subcore_axis_name="s")


_TB = 512


def _router_body(x_ref, gw_ref, wi_ref, ww_ref):
    x = x_ref[...]
    gw = gw_ref[...]
    logits = lax.dot_general(
        x, gw, (((1,), (1,)), ((), ())),
        preferred_element_type=F32)
    col = lax.broadcasted_iota(I32, (_TB, E), 1)
    m1 = jnp.max(logits, axis=1, keepdims=True)
    i1 = jnp.min(jnp.where(logits == m1, col, E), axis=1, keepdims=True)
    masked = jnp.where(col == i1, -jnp.inf, logits)
    m2 = jnp.max(masked, axis=1, keepdims=True)
    i2 = jnp.min(jnp.where(masked == m2, col, E), axis=1, keepdims=True)
    w1 = jax.nn.sigmoid(m1 - m2)
    wi_ref[...] = jnp.concatenate([i1, i2], axis=1)
    ww_ref[...] = jnp.concatenate([w1, 1.0 - w1], axis=1)


def _router(x, gate_weight):
    return pl.pallas_call(
        _router_body,
        grid=(T // _TB,),
        in_specs=[pl.BlockSpec((_TB, H), lambda b: (b, 0)),
                  pl.BlockSpec((E, H), lambda b: (0, 0))],
        out_specs=[pl.BlockSpec((_TB, K), lambda b: (b, 0)),
                   pl.BlockSpec((_TB, K), lambda b: (b, 0))],
        out_shape=[jax.ShapeDtypeStruct((T, K), I32),
                   jax.ShapeDtypeStruct((T, K), F32)],
    )(x, gate_weight)


_PG = 128
_NG = P // _PG


def _dispatch(topi):
    ef = topi.reshape(-1).astype(I32)
    eids = jnp.arange(E, dtype=I32)
    onehot = (ef[:, None] == eids[None, :]).astype(F32)
    counts = jnp.sum(onehot, axis=0)
    padded = jnp.ceil(counts / R) * R
    pad_end = jnp.cumsum(padded)
    pad_start = pad_end - padded
    mg = onehot.reshape(_NG, _PG, E)
    i_ = lax.broadcasted_iota(F32, (_PG, _PG), 0)
    j_ = lax.broadcasted_iota(F32, (_PG, _PG), 1)
    ltri = (j_ < i_).astype(F32)
    cin = jnp.einsum('ij,gje->gie', ltri, mg)
    gsum = jnp.sum(mg, axis=1)
    gpre = jnp.cumsum(gsum, axis=0) - gsum
    rank = jnp.sum((cin + gpre[:, None, :]) * mg, axis=2).reshape(P)
    base = jnp.sum(onehot * pad_start[None, :], axis=1)
    dest = (base + rank).astype(I32)
    bstarts = (jnp.arange(NB, dtype=F32) * R)
    block_expert = jnp.minimum(
        jnp.sum((pad_end[None, :] <= bstarts[:, None]).astype(I32), axis=1),
        E - 1).astype(I32)
    block_meta = jnp.where(bstarts < pad_end[E - 1], block_expert, -1)
    pos = dest.reshape(T, K)
    return block_meta.astype(I32), pos[:, 0], pos[:, 1]


_TPW = T // NW


@functools.cache
def _scatter_rows_kernel():
    @functools.partial(
        pl.kernel, mesh=_sc_mesh(),
        out_type=jax.ShapeDtypeStruct((NPAD, H), F32),
        scratch_types=[pltpu.VMEM((_TPW,), I32),
                       pltpu.VMEM((_TPW,), I32),
                       pltpu.VMEM((_TPW, H), F32),
                       pltpu.SemaphoreType.DMA])
    def _scatter_rows(x_hbm, p0_hbm, p1_hbm, out_hbm, idx0, idx1, buf_v, sem):
        wid = lax.axis_index("s") * NC + lax.axis_index("c")
        base = wid * _TPW
        pltpu.sync_copy(p0_hbm.at[pl.ds(base, _TPW)], idx0)
        pltpu.sync_copy(p1_hbm.at[pl.ds(base, _TPW)], idx1)
        pltpu.sync_copy(x_hbm.at[pl.ds(base, _TPW)], buf_v)
        pltpu.async_copy(buf_v, out_hbm.at[idx0], sem).wait()
        pltpu.async_copy(buf_v, out_hbm.at[idx1], sem).wait()

    return _scatter_rows


def _moe_body(meta_ref, x_ref, gu_ref, dn_ref, y_ref):
    @pl.when(meta_ref[pl.program_id(0)] >= 0)
    def _():
        x = x_ref[...].astype(BF16)
        z = lax.dot_general(
            x, gu_ref[0], (((1,), (1,)), ((), ())),
            preferred_element_type=F32)
        g = z[:, :FF]
        u = z[:, FF:]
        h = g * jax.nn.sigmoid(g) * u
        o = lax.dot_general(
            h.astype(BF16), dn_ref[0], (((1,), (1,)), ((), ())),
            preferred_element_type=F32)
        y_ref[...] = o


def _moe(block_meta, x_sorted, gate_up_proj, down_proj):
    grid_spec = pltpu.PrefetchScalarGridSpec(
        num_scalar_prefetch=1,
        grid=(NB,),
        in_specs=[
            pl.BlockSpec((R, H),
                         lambda b, m: (jnp.where(m[b] < 0, 0, b), 0)),
            pl.BlockSpec((1, 2 * FF, H),
                         lambda b, m: (jnp.maximum(m[b], 0), 0, 0)),
            pl.BlockSpec((1, H, FF),
                         lambda b, m: (jnp.maximum(m[b], 0), 0, 0)),
        ],
        out_specs=pl.BlockSpec((R, H), lambda b, m: (b, 0)),
    )
    return pl.pallas_call(
        _moe_body,
        grid_spec=grid_spec,
        out_shape=jax.ShapeDtypeStruct((NPAD, H), F32),
    )(block_meta, x_sorted, gate_up_proj, down_proj)


@functools.cache
def _combine_gather_kernel():
    @functools.partial(
        pl.kernel, mesh=_sc_mesh(),
        out_type=jax.ShapeDtypeStruct((2 * T, H), F32),
        scratch_types=[pltpu.VMEM((2 * _TPW,), I32),
                       pltpu.VMEM((2 * _TPW, H), F32),
                       pltpu.SemaphoreType.DMA])
    def _combine_gather(y_hbm, p0_hbm, p1_hbm, out_hbm, idx_v, buf_v, sem):
        wid = lax.axis_index("s") * NC + lax.axis_index("c")
        base = wid * _TPW
        pltpu.sync_copy(p0_hbm.at[pl.ds(base, _TPW)], idx_v.at[pl.ds(0, _TPW)])
        pltpu.sync_copy(p1_hbm.at[pl.ds(base, _TPW)],
                        idx_v.at[pl.ds(_TPW, _TPW)])
        pltpu.async_copy(y_hbm.at[idx_v], buf_v, sem).wait()
        pltpu.sync_copy(buf_v.at[pl.ds(0, _TPW)], out_hbm.at[pl.ds(base, _TPW)])
        pltpu.sync_copy(buf_v.at[pl.ds(_TPW, _TPW)],
                        out_hbm.at[pl.ds(T + base, _TPW)])

    return _combine_gather


def _add_body(a_ref, w_ref, o_ref):
    w = w_ref[...]
    o_ref[...] = a_ref[0] * w[:, 0:1] + a_ref[1] * w[:, 1:2]


def _add2(ycomb, topw):
    return pl.pallas_call(
        _add_body,
        grid=(T // 256,),
        in_specs=[pl.BlockSpec((2, 256, H), lambda b: (0, b, 0)),
                  pl.BlockSpec((256, K), lambda b: (b, 0))],
        out_specs=pl.BlockSpec((256, H), lambda b: (b, 0)),
        out_shape=jax.ShapeDtypeStruct((T, H), F32),
    )(ycomb, topw)


def kernel(layer_input, gate_weight, gate_up_proj, down_proj):
    b, s, h = layer_input.shape
    x = layer_input.reshape(-1, h)
    topi, topw = _router(x, gate_weight)
    block_expert, pos0, pos1 = _dispatch(topi)
    x_sorted = _scatter_rows_kernel()(x, pos0, pos1)
    y_sorted = _moe(block_expert, x_sorted,
                    gate_up_proj.astype(BF16), down_proj.astype(BF16))
    ycomb = _combine_gather_kernel()(y_sorted, pos0, pos1)
    out = _add2(ycomb.reshape(2, T, H), topw)
    return out.reshape(b, s, h)


# speedup vs baseline: 1.0933x; 1.0933x over previous
import functools

import jax
import jax.numpy as jnp
from jax import lax
from jax.experimental import pallas as pl
from jax.experimental.pallas import tpu as pltpu
from jax.experimental.pallas import tpu_sc as plsc

E = 64
K = 2
H = 768
FF = 128
T = 2048
P = T * K
R = 64
NB = P // R + E
NPAD = NB * R

NC, NS, L = 2, 16, 16
NW = NC * NS

F32 = jnp.float32
BF16 = jnp.bfloat16
I32 = jnp.int32

@functools.cache
def _sc_mesh():
    return plsc.VectorSubcoreMesh(core_axis_name="c", subcore_axis_name="s")


_TB = 512


def _router_body(x_ref, gw_ref, wi_ref, ww_ref):
    x = x_ref[...]
    gw = gw_ref[...]
    logits = lax.dot_general(
        x, gw, (((1,), (1,)), ((), ())),
        preferred_element_type=F32)
    col = lax.broadcasted_iota(I32, (_TB, E), 1)
    m1 = jnp.max(logits, axis=1, keepdims=True)
    i1 = jnp.min(jnp.where(logits == m1, col, E), axis=1, keepdims=True)
    masked = jnp.where(col == i1, -jnp.inf, logits)
    m2 = jnp.max(masked, axis=1, keepdims=True)
    i2 = jnp.min(jnp.where(masked == m2, col, E), axis=1, keepdims=True)
    w1 = jax.nn.sigmoid(m1 - m2)
    wi_ref[...] = jnp.concatenate([i1, i2], axis=1)
    ww_ref[...] = jnp.concatenate([w1, 1.0 - w1], axis=1)


def _router(x, gate_weight):
    return pl.pallas_call(
        _router_body,
        grid=(T // _TB,),
        in_specs=[pl.BlockSpec((_TB, H), lambda b: (b, 0)),
                  pl.BlockSpec((E, H), lambda b: (0, 0))],
        out_specs=[pl.BlockSpec((_TB, K), lambda b: (b, 0)),
                   pl.BlockSpec((_TB, K), lambda b: (b, 0))],
        out_shape=[jax.ShapeDtypeStruct((T, K), I32),
                   jax.ShapeDtypeStruct((T, K), F32)],
    )(x, gate_weight)


_PG = 128
_NG = P // _PG


def _dispatch(topi):
    ef = topi.reshape(-1).astype(I32)
    eids = jnp.arange(E, dtype=I32)
    onehot = (ef[:, None] == eids[None, :]).astype(F32)
    counts = jnp.sum(onehot, axis=0)
    padded = jnp.ceil(counts / R) * R
    pad_end = jnp.cumsum(padded)
    pad_start = pad_end - padded
    mg = onehot.reshape(_NG, _PG, E)
    i_ = lax.broadcasted_iota(F32, (_PG, _PG), 0)
    j_ = lax.broadcasted_iota(F32, (_PG, _PG), 1)
    ltri = (j_ < i_).astype(F32)
    cin = jnp.einsum('ij,gje->gie', ltri, mg)
    gsum = jnp.sum(mg, axis=1)
    gpre = jnp.cumsum(gsum, axis=0) - gsum
    rank = jnp.sum((cin + gpre[:, None, :]) * mg, axis=2).reshape(P)
    base = jnp.sum(onehot * pad_start[None, :], axis=1)
    dest = (base + rank).astype(I32)
    bstarts = (jnp.arange(NB, dtype=F32) * R)
    block_expert = jnp.minimum(
        jnp.sum((pad_end[None, :] <= bstarts[:, None]).astype(I32), axis=1),
        E - 1).astype(I32)
    block_meta = jnp.where(bstarts < pad_end[E - 1], block_expert, -1)
    pos = dest.reshape(T, K)
    return block_meta.astype(I32), pos[:, 0], pos[:, 1]


_TPW = T // NW


@functools.cache
def _scatter_rows_kernel():
    @functools.partial(
        pl.kernel, mesh=_sc_mesh(),
        out_type=jax.ShapeDtypeStruct((NPAD, H), F32),
        scratch_types=[pltpu.VMEM((_TPW,), I32),
                       pltpu.VMEM((_TPW,), I32),
                       pltpu.VMEM((_TPW, H), F32),
                       pltpu.SemaphoreType.DMA])
    def _scatter_rows(x_hbm, p0_hbm, p1_hbm, out_hbm, idx0, idx1, buf_v, sem):
        wid = lax.axis_index("s") * NC + lax.axis_index("c")
        base = wid * _TPW
        pltpu.sync_copy(p0_hbm.at[pl.ds(base, _TPW)], idx0)
        pltpu.sync_copy(p1_hbm.at[pl.ds(base, _TPW)], idx1)
        pltpu.sync_copy(x_hbm.at[pl.ds(base, _TPW)], buf_v)
        pltpu.async_copy(buf_v, out_hbm.at[idx0], sem).wait()
        pltpu.async_copy(buf_v, out_hbm.at[idx1], sem).wait()

    return _scatter_rows


def _moe_body(meta_ref, x_ref, gu_ref, dn_ref, y_ref):
    @pl.when(meta_ref[pl.program_id(0)] >= 0)
    def _():
        x = x_ref[...]
        z = lax.dot_general(
            x, gu_ref[0], (((1,), (1,)), ((), ())),
            preferred_element_type=F32)
        g = z[:, :FF]
        u = z[:, FF:]
        h = g * jax.nn.sigmoid(g) * u
        o = lax.dot_general(
            h, dn_ref[0], (((1,), (1,)), ((), ())),
            preferred_element_type=F32)
        y_ref[...] = o


def _moe(block_meta, x_sorted, gate_up_proj, down_proj):
    grid_spec = pltpu.PrefetchScalarGridSpec(
        num_scalar_prefetch=1,
        grid=(NB,),
        in_specs=[
            pl.BlockSpec((R, H),
                         lambda b, m: (jnp.where(m[b] < 0, 0, b), 0)),
            pl.BlockSpec((1, 2 * FF, H),
                         lambda b, m: (jnp.maximum(m[b], 0), 0, 0)),
            pl.BlockSpec((1, H, FF),
                         lambda b, m: (jnp.maximum(m[b], 0), 0, 0)),
        ],
        out_specs=pl.BlockSpec(
            (R, H), lambda b, m: (jnp.where(m[b] < 0, NB - 1, b), 0)),
    )
    return pl.pallas_call(
        _moe_body,
        grid_spec=grid_spec,
        out_shape=jax.ShapeDtypeStruct((NPAD, H), F32),
    )(block_meta, x_sorted, gate_up_proj, down_proj)


@functools.cache
def _combine_gather_kernel():
    @functools.partial(
        pl.kernel, mesh=_sc_mesh(),
        out_type=jax.ShapeDtypeStruct((2 * T, H), F32),
        scratch_types=[pltpu.VMEM((2 * _TPW,), I32),
                       pltpu.VMEM((2 * _TPW, H), F32),
                       pltpu.SemaphoreType.DMA])
    def _combine_gather(y_hbm, p0_hbm, p1_hbm, out_hbm, idx_v, buf_v, sem):
        wid = lax.axis_index("s") * NC + lax.axis_index("c")
        base = wid * _TPW
        pltpu.sync_copy(p0_hbm.at[pl.ds(base, _TPW)], idx_v.at[pl.ds(0, _TPW)])
        pltpu.sync_copy(p1_hbm.at[pl.ds(base, _TPW)],
                        idx_v.at[pl.ds(_TPW, _TPW)])
        pltpu.async_copy(y_hbm.at[idx_v], buf_v, sem).wait()
        pltpu.sync_copy(buf_v.at[pl.ds(0, _TPW)], out_hbm.at[pl.ds(base, _TPW)])
        pltpu.sync_copy(buf_v.at[pl.ds(_TPW, _TPW)],
                        out_hbm.at[pl.ds(T + base, _TPW)])

    return _combine_gather


def _add_body(a_ref, w_ref, o_ref):
    w = w_ref[...]
    o_ref[...] = a_ref[0] * w[:, 0:1] + a_ref[1] * w[:, 1:2]


def _add2(ycomb, topw):
    return pl.pallas_call(
        _add_body,
        grid=(T // 256,),
        in_specs=[pl.BlockSpec((2, 256, H), lambda b: (0, b, 0)),
                  pl.BlockSpec((256, K), lambda b: (b, 0))],
        out_specs=pl.BlockSpec((256, H), lambda b: (b, 0)),
        out_shape=jax.ShapeDtypeStruct((T, H), F32),
    )(ycomb, topw)


def kernel(layer_input, gate_weight, gate_up_proj, down_proj):
    b, s, h = layer_input.shape
    x = layer_input.reshape(-1, h)
    topi, topw = _router(x, gate_weight)
    block_expert, pos0, pos1 = _dispatch(topi)
    x_sorted = _scatter_rows_kernel()(x, pos0, pos1)
    y_sorted = _moe(block_expert, x_sorted, gate_up_proj, down_proj)
    ycomb = _combine_gather_kernel()(y_sorted, pos0, pos1)
    out = _add2(ycomb.reshape(2, T, H), topw)
    return out.reshape(b, s, h)


# speedup vs baseline: 1.2655x; 1.1575x over previous
import functools

import jax
import jax.numpy as jnp
from jax import lax
from jax.experimental import pallas as pl
from jax.experimental.pallas import tpu as pltpu
from jax.experimental.pallas import tpu_sc as plsc

E = 64
K = 2
H = 768
FF = 128
T = 2048
P = T * K
R = 128
NB = P // R + E
NPAD = NB * R

NC, NS, L = 2, 16, 16
NW = NC * NS

F32 = jnp.float32
BF16 = jnp.bfloat16
I32 = jnp.int32

@functools.cache
def _sc_mesh():
    return plsc.VectorSubcoreMesh(core_axis_name="c", subcore_axis_name="s")


_TB = 512


def _router_body(x_ref, gw_ref, wi_ref, ww_ref):
    x = x_ref[...]
    gw = gw_ref[...]
    logits = lax.dot_general(
        x, gw, (((1,), (1,)), ((), ())),
        preferred_element_type=F32)
    col = lax.broadcasted_iota(I32, (_TB, E), 1)
    m1 = jnp.max(logits, axis=1, keepdims=True)
    i1 = jnp.min(jnp.where(logits == m1, col, E), axis=1, keepdims=True)
    masked = jnp.where(col == i1, -jnp.inf, logits)
    m2 = jnp.max(masked, axis=1, keepdims=True)
    i2 = jnp.min(jnp.where(masked == m2, col, E), axis=1, keepdims=True)
    w1 = jax.nn.sigmoid(m1 - m2)
    wi_ref[...] = jnp.concatenate([i1, i2], axis=1)
    ww_ref[...] = jnp.concatenate([w1, 1.0 - w1], axis=1)


def _router(x, gate_weight):
    return pl.pallas_call(
        _router_body,
        grid=(T // _TB,),
        in_specs=[pl.BlockSpec((_TB, H), lambda b: (b, 0)),
                  pl.BlockSpec((E, H), lambda b: (0, 0))],
        out_specs=[pl.BlockSpec((_TB, K), lambda b: (b, 0)),
                   pl.BlockSpec((_TB, K), lambda b: (b, 0))],
        out_shape=[jax.ShapeDtypeStruct((T, K), I32),
                   jax.ShapeDtypeStruct((T, K), F32)],
    )(x, gate_weight)


_PG = 128
_NG = P // _PG


def _dispatch(topi):
    ef = topi.reshape(-1).astype(I32)
    eids = jnp.arange(E, dtype=I32)
    onehot = (ef[:, None] == eids[None, :]).astype(F32)
    counts = jnp.sum(onehot, axis=0)
    padded = jnp.ceil(counts / R) * R
    pad_end = jnp.cumsum(padded)
    pad_start = pad_end - padded
    mg = onehot.reshape(_NG, _PG, E)
    i_ = lax.broadcasted_iota(F32, (_PG, _PG), 0)
    j_ = lax.broadcasted_iota(F32, (_PG, _PG), 1)
    ltri = (j_ < i_).astype(F32)
    cin = jnp.einsum('ij,gje->gie', ltri, mg)
    gsum = jnp.sum(mg, axis=1)
    gpre = jnp.cumsum(gsum, axis=0) - gsum
    rank = jnp.sum((cin + gpre[:, None, :]) * mg, axis=2).reshape(P)
    base = jnp.sum(onehot * pad_start[None, :], axis=1)
    dest = (base + rank).astype(I32)
    bstarts = (jnp.arange(NB, dtype=F32) * R)
    block_expert = jnp.minimum(
        jnp.sum((pad_end[None, :] <= bstarts[:, None]).astype(I32), axis=1),
        E - 1).astype(I32)
    block_meta = jnp.where(bstarts < pad_end[E - 1], block_expert, -1)
    pos = dest.reshape(T, K)
    return block_meta.astype(I32), pos[:, 0], pos[:, 1]


_TPW = T // NW


@functools.cache
def _scatter_rows_kernel():
    @functools.partial(
        pl.kernel, mesh=_sc_mesh(),
        out_type=jax.ShapeDtypeStruct((NPAD, H), F32),
        scratch_types=[pltpu.VMEM((_TPW,), I32),
                       pltpu.VMEM((_TPW,), I32),
                       pltpu.VMEM((_TPW, H), F32),
                       pltpu.SemaphoreType.DMA])
    def _scatter_rows(x_hbm, p0_hbm, p1_hbm, out_hbm, idx0, idx1, buf_v, sem):
        wid = lax.axis_index("s") * NC + lax.axis_index("c")
        base = wid * _TPW
        pltpu.sync_copy(p0_hbm.at[pl.ds(base, _TPW)], idx0)
        pltpu.sync_copy(p1_hbm.at[pl.ds(base, _TPW)], idx1)
        pltpu.sync_copy(x_hbm.at[pl.ds(base, _TPW)], buf_v)
        pltpu.async_copy(buf_v, out_hbm.at[idx0], sem).wait()
        pltpu.async_copy(buf_v, out_hbm.at[idx1], sem).wait()

    return _scatter_rows


def _moe_body(meta_ref, x_ref, gu_ref, dn_ref, y_ref):
    @pl.when(meta_ref[pl.program_id(0)] >= 0)
    def _():
        x = x_ref[...]
        z = lax.dot_general(
            x, gu_ref[0], (((1,), (1,)), ((), ())),
            preferred_element_type=F32)
        g = z[:, :FF]
        u = z[:, FF:]
        h = g * jax.nn.sigmoid(g) * u
        o = lax.dot_general(
            h, dn_ref[0], (((1,), (1,)), ((), ())),
            preferred_element_type=F32)
        y_ref[...] = o


def _moe(block_meta, x_sorted, gate_up_proj, down_proj):
    grid_spec = pltpu.PrefetchScalarGridSpec(
        num_scalar_prefetch=1,
        grid=(NB,),
        in_specs=[
            pl.BlockSpec((R, H),
                         lambda b, m: (jnp.where(m[b] < 0, 0, b), 0)),
            pl.BlockSpec((1, 2 * FF, H),
                         lambda b, m: (jnp.maximum(m[b], 0), 0, 0)),
            pl.BlockSpec((1, H, FF),
                         lambda b, m: (jnp.maximum(m[b], 0), 0, 0)),
        ],
        out_specs=pl.BlockSpec(
            (R, H), lambda b, m: (jnp.where(m[b] < 0, NB - 1, b), 0)),
    )
    return pl.pallas_call(
        _moe_body,
        grid_spec=grid_spec,
        out_shape=jax.ShapeDtypeStruct((NPAD, H), F32),
    )(block_meta, x_sorted, gate_up_proj, down_proj)


@functools.cache
def _combine_gather_kernel():
    @functools.partial(
        pl.kernel, mesh=_sc_mesh(),
        out_type=jax.ShapeDtypeStruct((2 * T, H), F32),
        scratch_types=[pltpu.VMEM((2 * _TPW,), I32),
                       pltpu.VMEM((2 * _TPW, H), F32),
                       pltpu.SemaphoreType.DMA])
    def _combine_gather(y_hbm, p0_hbm, p1_hbm, out_hbm, idx_v, buf_v, sem):
        wid = lax.axis_index("s") * NC + lax.axis_index("c")
        base = wid * _TPW
        pltpu.sync_copy(p0_hbm.at[pl.ds(base, _TPW)], idx_v.at[pl.ds(0, _TPW)])
        pltpu.sync_copy(p1_hbm.at[pl.ds(base, _TPW)],
                        idx_v.at[pl.ds(_TPW, _TPW)])
        pltpu.async_copy(y_hbm.at[idx_v], buf_v, sem).wait()
        pltpu.sync_copy(buf_v.at[pl.ds(0, _TPW)], out_hbm.at[pl.ds(base, _TPW)])
        pltpu.sync_copy(buf_v.at[pl.ds(_TPW, _TPW)],
                        out_hbm.at[pl.ds(T + base, _TPW)])

    return _combine_gather


def _add_body(a_ref, w_ref, o_ref):
    w = w_ref[...]
    o_ref[...] = a_ref[0] * w[:, 0:1] + a_ref[1] * w[:, 1:2]


def _add2(ycomb, topw):
    return pl.pallas_call(
        _add_body,
        grid=(T // 256,),
        in_specs=[pl.BlockSpec((2, 256, H), lambda b: (0, b, 0)),
                  pl.BlockSpec((256, K), lambda b: (b, 0))],
        out_specs=pl.BlockSpec((256, H), lambda b: (b, 0)),
        out_shape=jax.ShapeDtypeStruct((T, H), F32),
    )(ycomb, topw)


def kernel(layer_input, gate_weight, gate_up_proj, down_proj):
    b, s, h = layer_input.shape
    x = layer_input.reshape(-1, h)
    topi, topw = _router(x, gate_weight)
    block_expert, pos0, pos1 = _dispatch(topi)
    x_sorted = _scatter_rows_kernel()(x, pos0, pos1)
    y_sorted = _moe(block_expert, x_sorted, gate_up_proj, down_proj)
    ycomb = _combine_gather_kernel()(y_sorted, pos0, pos1)
    out = _add2(ycomb.reshape(2, T, H), topw)
    return out.reshape(b, s, h)
